# baseline (speedup 1.0000x reference)
import functools

import jax
import jax.numpy as jnp
from jax import lax
from jax.experimental import pallas as pl
from jax.experimental.pallas import tpu as pltpu
from jax.experimental.pallas import tpu_sc as plsc

NUM_WORKERS = 32
CHUNK = 128
NBUF = 4


def _gather_kernel(n_chunks, chunk, d, per_w):
    mesh = plsc.VectorSubcoreMesh(core_axis_name="c", subcore_axis_name="s")

    @functools.partial(
        pl.kernel,
        mesh=mesh,
        out_type=jax.ShapeDtypeStruct((NUM_WORKERS * per_w, 2 * d), jnp.float32),
        scratch_types=(
            [pltpu.VMEM((n_chunks, chunk), jnp.int32),
             pltpu.VMEM((NBUF, chunk, 2 * d), jnp.float32)]
            + [pltpu.SemaphoreType.DMA] * (2 * NBUF)
        ),
        compiler_params=pltpu.CompilerParams(use_tc_tiling_on_sc=False),
    )
    def emb(idx_hbm, tab_hbm, out_hbm, idx_v, rows_v, *sems):
        gsem = sems[:NBUF]
        wsem = sems[NBUF:]
        c = lax.axis_index("c")
        s = lax.axis_index("s")
        wid = s * 2 + c
        base = wid * per_w
        pltpu.sync_copy(idx_hbm.at[wid], idx_v)

        def fire_gather(slot, b):
            pltpu.async_copy(tab_hbm.at[idx_v.at[slot]], rows_v.at[b], gsem[b])

        def wait_gather(slot, b):
            pltpu.make_async_copy(
                tab_hbm.at[idx_v.at[slot]], rows_v.at[b], gsem[b]).wait()

        def fire_write(slot, b):
            pltpu.async_copy(
                rows_v.at[b, :, pl.ds(0, d)],
                out_hbm.at[pl.ds(base + slot * chunk, chunk), pl.ds(0, d)],
                wsem[b])

        def wait_write(slot, b):
            pltpu.make_async_copy(
                rows_v.at[b, :, pl.ds(0, d)],
                out_hbm.at[pl.ds(base + slot * chunk, chunk), pl.ds(0, d)],
                wsem[b]).wait()

        def do_slot(slot, k, fire, wait_prev):
            b = k % NBUF
            pb = (k - 1) % NBUF
            wait_gather(slot, b)
            fire_write(slot, b)
            if wait_prev:
                wait_write(slot - 1, pb)
            if fire:
                fire_gather(slot + NBUF - 1, pb)

        for j in range(NBUF - 1):
            fire_gather(j, j)

        for k in range(NBUF):
            do_slot(k, k, fire=(k + NBUF - 1 < n_chunks), wait_prev=(k >= 1))

        n_rounds = n_chunks // NBUF

        def body(r, _):
            s0 = r * NBUF
            for k in range(NBUF):
                do_slot(s0 + k, k, fire=True, wait_prev=True)
            return 0

        lax.fori_loop(1, n_rounds - 1, body, 0)

        s0 = (n_rounds - 1) * NBUF
        for k in range(NBUF):
            do_slot(s0 + k, k, fire=(s0 + k + NBUF - 1 < n_chunks),
                    wait_prev=True)

        wait_write(n_chunks - 1, (n_chunks - 1) % NBUF)

    return emb


def kernel(indices, table):
    b, sq = indices.shape
    v, d = table.shape
    n = b * sq
    per_w = n // NUM_WORKERS
    n_chunks = per_w // CHUNK
    idx = indices.reshape(NUM_WORKERS, n_chunks, CHUNK).astype(jnp.int32)
    tab128 = jnp.pad(table, ((0, 0), (0, d)))
    out = _gather_kernel(n_chunks, CHUNK, d, per_w)(idx, tab128)
    return out[:, :d].reshape(b, sq, d)


# speedup vs baseline: 1.5462x; 1.5462x over previous
import functools

import jax
import jax.numpy as jnp
from jax import lax
from jax.experimental import pallas as pl
from jax.experimental.pallas import tpu as pltpu
from jax.experimental.pallas import tpu_sc as plsc

NUM_WORKERS = 32
CHUNK = 128
NBUF = 4


def _gather_kernel(n_chunks, chunk, d, per_w):
    mesh = plsc.VectorSubcoreMesh(core_axis_name="c", subcore_axis_name="s")

    @functools.partial(
        pl.kernel,
        mesh=mesh,
        out_type=jax.ShapeDtypeStruct((NUM_WORKERS * per_w, 2 * d), jnp.float32),
        scratch_types=(
            [pltpu.VMEM((n_chunks, chunk), jnp.int32),
             pltpu.VMEM((NBUF, chunk, d), jnp.float32)]
            + [pltpu.SemaphoreType.DMA] * (2 * NBUF)
        ),
        compiler_params=pltpu.CompilerParams(use_tc_tiling_on_sc=False),
    )
    def emb(idx_hbm, tab_hbm, out_hbm, idx_v, rows_v, *sems):
        gsem = sems[:NBUF]
        wsem = sems[NBUF:]
        c = lax.axis_index("c")
        s = lax.axis_index("s")
        wid = s * 2 + c
        base = wid * per_w
        pltpu.sync_copy(idx_hbm.at[wid], idx_v)

        def fire_gather(slot, b):
            pltpu.async_copy(tab_hbm.at[idx_v.at[slot]], rows_v.at[b], gsem[b])

        def wait_gather(slot, b):
            pltpu.make_async_copy(
                tab_hbm.at[idx_v.at[slot]], rows_v.at[b], gsem[b]).wait()

        def fire_write(slot, b):
            pltpu.async_copy(
                rows_v.at[b],
                out_hbm.at[pl.ds(base + slot * chunk, chunk), pl.ds(0, d)],
                wsem[b])

        def wait_write(slot, b):
            pltpu.make_async_copy(
                rows_v.at[b],
                out_hbm.at[pl.ds(base + slot * chunk, chunk), pl.ds(0, d)],
                wsem[b]).wait()

        def do_slot(slot, k, fire, wait_prev):
            b = k % NBUF
            pb = (k - 1) % NBUF
            wait_gather(slot, b)
            fire_write(slot, b)
            if wait_prev:
                wait_write(slot - 1, pb)
            if fire:
                fire_gather(slot + NBUF - 1, pb)

        for j in range(NBUF - 1):
            fire_gather(j, j)

        for k in range(NBUF):
            do_slot(k, k, fire=(k + NBUF - 1 < n_chunks), wait_prev=(k >= 1))

        n_rounds = n_chunks // NBUF

        def body(r, _):
            s0 = r * NBUF
            for k in range(NBUF):
                do_slot(s0 + k, k, fire=True, wait_prev=True)
            return 0

        lax.fori_loop(1, n_rounds - 1, body, 0)

        s0 = (n_rounds - 1) * NBUF
        for k in range(NBUF):
            do_slot(s0 + k, k, fire=(s0 + k + NBUF - 1 < n_chunks),
                    wait_prev=True)

        wait_write(n_chunks - 1, (n_chunks - 1) % NBUF)

    return emb


TBLK = 8192
THALF = TBLK // 2


def _transpose_body(tt_ref, out_ref):
    x = tt_ref[...]
    y1 = jnp.swapaxes(x[:, :THALF], 0, 1)
    y2 = jnp.swapaxes(x[:, THALF:], 0, 1)
    out_ref[...] = jnp.concatenate([y1, y2], axis=1)


def _table_rowmajor(tt, v, d):
    grid = (v + TBLK - 1) // TBLK
    return pl.pallas_call(
        _transpose_body,
        grid=(grid,),
        in_specs=[pl.BlockSpec((d, TBLK), lambda j: (0, j))],
        out_specs=pl.BlockSpec((THALF, 2 * d), lambda j: (j, 0)),
        out_shape=jax.ShapeDtypeStruct((grid * THALF, 2 * d), jnp.float32),
    )(tt)


def kernel(indices, table):
    b, sq = indices.shape
    v, d = table.shape
    n = b * sq
    per_w = n // NUM_WORKERS
    n_chunks = per_w // CHUNK
    idx = indices.astype(jnp.int32)
    t = idx % TBLK
    idx2 = (idx - t) + 2 * (t % THALF) + t // THALF
    idx2 = idx2.reshape(NUM_WORKERS, n_chunks, CHUNK)
    tab2 = _table_rowmajor(table.T, v, d)
    tab_rm = tab2.reshape(2 * tab2.shape[0], d)
    out = _gather_kernel(n_chunks, CHUNK, d, per_w)(idx2, tab_rm)
    return out[:, :d].reshape(b, sq, d)


# speedup vs baseline: 1.6306x; 1.0546x over previous
import functools

import jax
import jax.numpy as jnp
from jax import lax
from jax.experimental import pallas as pl
from jax.experimental.pallas import tpu as pltpu
from jax.experimental.pallas import tpu_sc as plsc

NUM_WORKERS = 32
CHUNK = 128
NBUF = 4


def _gather_kernel(n_chunks, chunk, d, per_w):
    mesh = plsc.VectorSubcoreMesh(core_axis_name="c", subcore_axis_name="s")

    @functools.partial(
        pl.kernel,
        mesh=mesh,
        out_type=jax.ShapeDtypeStruct((NUM_WORKERS * per_w, 2 * d), jnp.float32),
        scratch_types=(
            [pltpu.VMEM((n_chunks, chunk), jnp.int32),
             pltpu.VMEM((NBUF, chunk, d), jnp.float32)]
            + [pltpu.SemaphoreType.DMA] * (2 * NBUF)
        ),
        compiler_params=pltpu.CompilerParams(use_tc_tiling_on_sc=False),
    )
    def emb(idx_hbm, tab_hbm, out_hbm, idx_v, rows_v, *sems):
        gsem = sems[:NBUF]
        wsem = sems[NBUF:]
        c = lax.axis_index("c")
        s = lax.axis_index("s")
        wid = s * 2 + c
        base = wid * per_w
        pltpu.sync_copy(idx_hbm.at[wid], idx_v)

        def fire_gather(slot, b):
            pltpu.async_copy(tab_hbm.at[idx_v.at[slot]], rows_v.at[b], gsem[b])

        def wait_gather(slot, b):
            pltpu.make_async_copy(
                tab_hbm.at[idx_v.at[slot]], rows_v.at[b], gsem[b]).wait()

        def fire_write(slot, b):
            pltpu.async_copy(
                rows_v.at[b],
                out_hbm.at[pl.ds(base + slot * chunk, chunk), pl.ds(0, d)],
                wsem[b])

        def wait_write(slot, b):
            pltpu.make_async_copy(
                rows_v.at[b],
                out_hbm.at[pl.ds(base + slot * chunk, chunk), pl.ds(0, d)],
                wsem[b]).wait()

        def do_slot(slot, k, fire, wait_prev):
            b = k % NBUF
            pb = (k - 1) % NBUF
            wait_gather(slot, b)
            fire_write(slot, b)
            if wait_prev:
                wait_write(slot - 1, pb)
            if fire:
                fire_gather(slot + NBUF - 1, pb)

        for j in range(NBUF - 1):
            fire_gather(j, j)

        for k in range(NBUF):
            do_slot(k, k, fire=(k + NBUF - 1 < n_chunks), wait_prev=(k >= 1))

        n_rounds = n_chunks // NBUF

        def body(r, _):
            s0 = r * NBUF
            for k in range(NBUF):
                do_slot(s0 + k, k, fire=True, wait_prev=True)
            return 0

        lax.fori_loop(1, n_rounds - 1, body, 0)

        s0 = (n_rounds - 1) * NBUF
        for k in range(NBUF):
            do_slot(s0 + k, k, fire=(s0 + k + NBUF - 1 < n_chunks),
                    wait_prev=True)

        wait_write(n_chunks - 1, (n_chunks - 1) % NBUF)

    return emb


TBLK = 16384
THALF = TBLK // 2


def _transpose_body(tt_ref, out_ref):
    x = tt_ref[...]
    y1 = jnp.swapaxes(x[:, :THALF], 0, 1)
    y2 = jnp.swapaxes(x[:, THALF:], 0, 1)
    out_ref[...] = jnp.concatenate([y1, y2], axis=1)


def _table_rowmajor(tt, v, d):
    grid = (v + TBLK - 1) // TBLK
    return pl.pallas_call(
        _transpose_body,
        grid=(grid,),
        in_specs=[pl.BlockSpec((d, TBLK), lambda j: (0, j))],
        out_specs=pl.BlockSpec((THALF, 2 * d), lambda j: (j, 0)),
        out_shape=jax.ShapeDtypeStruct((grid * THALF, 2 * d), jnp.float32),
    )(tt)


def kernel(indices, table):
    b, sq = indices.shape
    v, d = table.shape
    n = b * sq
    per_w = n // NUM_WORKERS
    n_chunks = per_w // CHUNK
    idx = indices.astype(jnp.int32)
    t = idx % TBLK
    idx2 = (idx - t) + 2 * (t % THALF) + t // THALF
    idx2 = idx2.reshape(NUM_WORKERS, n_chunks, CHUNK)
    tab2 = _table_rowmajor(table.T, v, d)
    tab_rm = tab2.reshape(2 * tab2.shape[0], d)
    out = _gather_kernel(n_chunks, CHUNK, d, per_w)(idx2, tab_rm)
    return out[:, :d].reshape(b, sq, d)


# speedup vs baseline: 1.6722x; 1.0255x over previous
import functools

import jax
import jax.numpy as jnp
from jax import lax
from jax.experimental import pallas as pl
from jax.experimental.pallas import tpu as pltpu
from jax.experimental.pallas import tpu_sc as plsc

NUM_WORKERS = 32
CHUNK = 128
NBUF = 4


def _gather_kernel(n_chunks, chunk, d, per_w):
    mesh = plsc.VectorSubcoreMesh(core_axis_name="c", subcore_axis_name="s")

    @functools.partial(
        pl.kernel,
        mesh=mesh,
        out_type=jax.ShapeDtypeStruct((NUM_WORKERS * per_w, 2 * d), jnp.float32),
        scratch_types=(
            [pltpu.VMEM((n_chunks, chunk), jnp.int32),
             pltpu.VMEM((NBUF, chunk, d), jnp.float32)]
            + [pltpu.SemaphoreType.DMA] * (2 * NBUF)
        ),
        compiler_params=pltpu.CompilerParams(use_tc_tiling_on_sc=False),
    )
    def emb(idx_hbm, tab_hbm, out_hbm, idx_v, rows_v, *sems):
        gsem = sems[:NBUF]
        wsem = sems[NBUF:]
        c = lax.axis_index("c")
        s = lax.axis_index("s")
        wid = s * 2 + c
        base = wid * per_w
        pltpu.sync_copy(idx_hbm.at[wid], idx_v)

        def fire_gather(slot, b):
            pltpu.async_copy(tab_hbm.at[idx_v.at[slot]], rows_v.at[b], gsem[b])

        def wait_gather(slot, b):
            pltpu.make_async_copy(
                tab_hbm.at[idx_v.at[slot]], rows_v.at[b], gsem[b]).wait()

        def fire_write(slot, b):
            pltpu.async_copy(
                rows_v.at[b],
                out_hbm.at[pl.ds(base + slot * chunk, chunk), pl.ds(0, d)],
                wsem[b])

        def wait_write(slot, b):
            pltpu.make_async_copy(
                rows_v.at[b],
                out_hbm.at[pl.ds(base + slot * chunk, chunk), pl.ds(0, d)],
                wsem[b]).wait()

        def do_slot(slot, k, fire, wait_prev):
            b = k % NBUF
            pb = (k - 1) % NBUF
            wait_gather(slot, b)
            fire_write(slot, b)
            if wait_prev:
                wait_write(slot - 1, pb)
            if fire:
                fire_gather(slot + NBUF - 1, pb)

        for j in range(NBUF - 1):
            fire_gather(j, j)

        for k in range(NBUF):
            do_slot(k, k, fire=(k + NBUF - 1 < n_chunks), wait_prev=(k >= 1))

        n_rounds = n_chunks // NBUF

        def body(r, _):
            s0 = r * NBUF
            for k in range(NBUF):
                do_slot(s0 + k, k, fire=True, wait_prev=True)
            return 0

        lax.fori_loop(1, n_rounds - 1, body, 0)

        s0 = (n_rounds - 1) * NBUF
        for k in range(NBUF):
            do_slot(s0 + k, k, fire=(s0 + k + NBUF - 1 < n_chunks),
                    wait_prev=True)

        wait_write(n_chunks - 1, (n_chunks - 1) % NBUF)

    return emb


TBLK = 32768
THALF = TBLK // 2


def _transpose_body(tt_ref, out_ref):
    x = tt_ref[...]
    y1 = jnp.swapaxes(x[:, :THALF], 0, 1)
    y2 = jnp.swapaxes(x[:, THALF:], 0, 1)
    out_ref[...] = jnp.concatenate([y1, y2], axis=1)


def _table_rowmajor(tt, v, d):
    grid = (v + TBLK - 1) // TBLK
    return pl.pallas_call(
        _transpose_body,
        grid=(grid,),
        in_specs=[pl.BlockSpec((d, TBLK), lambda j: (0, j))],
        out_specs=pl.BlockSpec((THALF, 2 * d), lambda j: (j, 0)),
        out_shape=jax.ShapeDtypeStruct((grid * THALF, 2 * d), jnp.float32),
        compiler_params=pltpu.CompilerParams(vmem_limit_bytes=100 * 1024 * 1024),
    )(tt)


def kernel(indices, table):
    b, sq = indices.shape
    v, d = table.shape
    n = b * sq
    per_w = n // NUM_WORKERS
    n_chunks = per_w // CHUNK
    idx = indices.astype(jnp.int32)
    t = idx % TBLK
    idx2 = (idx - t) + 2 * (t % THALF) + t // THALF
    idx2 = idx2.reshape(NUM_WORKERS, n_chunks, CHUNK)
    tab2 = _table_rowmajor(table.T, v, d)
    tab_rm = tab2.reshape(2 * tab2.shape[0], d)
    out = _gather_kernel(n_chunks, CHUNK, d, per_w)(idx2, tab_rm)
    return out[:, :d].reshape(b, sq, d)


# speedup vs baseline: 1.6757x; 1.0021x over previous
import functools

import jax
import jax.numpy as jnp
from jax import lax
from jax.experimental import pallas as pl
from jax.experimental.pallas import tpu as pltpu
from jax.experimental.pallas import tpu_sc as plsc

NUM_WORKERS = 32
CHUNK = 128
NBUF = 8


def _gather_kernel(n_chunks, chunk, d, per_w):
    mesh = plsc.VectorSubcoreMesh(core_axis_name="c", subcore_axis_name="s")

    @functools.partial(
        pl.kernel,
        mesh=mesh,
        out_type=jax.ShapeDtypeStruct((NUM_WORKERS * per_w, 2 * d), jnp.float32),
        scratch_types=(
            [pltpu.VMEM((n_chunks, chunk), jnp.int32),
             pltpu.VMEM((NBUF, chunk, d), jnp.float32)]
            + [pltpu.SemaphoreType.DMA] * (2 * NBUF)
        ),
        compiler_params=pltpu.CompilerParams(use_tc_tiling_on_sc=False),
    )
    def emb(idx_hbm, tab_hbm, out_hbm, idx_v, rows_v, *sems):
        gsem = sems[:NBUF]
        wsem = sems[NBUF:]
        c = lax.axis_index("c")
        s = lax.axis_index("s")
        wid = s * 2 + c
        base = wid * per_w
        pltpu.sync_copy(idx_hbm.at[wid], idx_v)

        def fire_gather(slot, b):
            pltpu.async_copy(tab_hbm.at[idx_v.at[slot]], rows_v.at[b], gsem[b])

        def wait_gather(slot, b):
            pltpu.make_async_copy(
                tab_hbm.at[idx_v.at[slot]], rows_v.at[b], gsem[b]).wait()

        def fire_write(slot, b):
            pltpu.async_copy(
                rows_v.at[b],
                out_hbm.at[pl.ds(base + slot * chunk, chunk), pl.ds(0, d)],
                wsem[b])

        def wait_write(slot, b):
            pltpu.make_async_copy(
                rows_v.at[b],
                out_hbm.at[pl.ds(base + slot * chunk, chunk), pl.ds(0, d)],
                wsem[b]).wait()

        def do_slot(slot, k, fire, wait_prev):
            b = k % NBUF
            pb = (k - 1) % NBUF
            wait_gather(slot, b)
            fire_write(slot, b)
            if wait_prev:
                wait_write(slot - 1, pb)
            if fire:
                fire_gather(slot + NBUF - 1, pb)

        for j in range(NBUF - 1):
            fire_gather(j, j)

        for k in range(NBUF):
            do_slot(k, k, fire=(k + NBUF - 1 < n_chunks), wait_prev=(k >= 1))

        n_rounds = n_chunks // NBUF

        def body(r, _):
            s0 = r * NBUF
            for k in range(NBUF):
                do_slot(s0 + k, k, fire=True, wait_prev=True)
            return 0

        lax.fori_loop(1, n_rounds - 1, body, 0)

        s0 = (n_rounds - 1) * NBUF
        for k in range(NBUF):
            do_slot(s0 + k, k, fire=(s0 + k + NBUF - 1 < n_chunks),
                    wait_prev=True)

        wait_write(n_chunks - 1, (n_chunks - 1) % NBUF)

    return emb


TBLK = 32768
THALF = TBLK // 2


def _transpose_body(tt_ref, out_ref):
    x = tt_ref[...]
    y1 = jnp.swapaxes(x[:, :THALF], 0, 1)
    y2 = jnp.swapaxes(x[:, THALF:], 0, 1)
    out_ref[...] = jnp.concatenate([y1, y2], axis=1)


def _table_rowmajor(tt, v, d):
    grid = (v + TBLK - 1) // TBLK
    return pl.pallas_call(
        _transpose_body,
        grid=(grid,),
        in_specs=[pl.BlockSpec((d, TBLK), lambda j: (0, j))],
        out_specs=pl.BlockSpec((THALF, 2 * d), lambda j: (j, 0)),
        out_shape=jax.ShapeDtypeStruct((grid * THALF, 2 * d), jnp.float32),
        compiler_params=pltpu.CompilerParams(vmem_limit_bytes=100 * 1024 * 1024),
    )(tt)


def kernel(indices, table):
    b, sq = indices.shape
    v, d = table.shape
    n = b * sq
    per_w = n // NUM_WORKERS
    n_chunks = per_w // CHUNK
    idx = indices.astype(jnp.int32)
    t = idx % TBLK
    idx2 = (idx - t) + 2 * (t % THALF) + t // THALF
    idx2 = idx2.reshape(NUM_WORKERS, n_chunks, CHUNK)
    tab2 = _table_rowmajor(table.T, v, d)
    tab_rm = tab2.reshape(2 * tab2.shape[0], d)
    out = _gather_kernel(n_chunks, CHUNK, d, per_w)(idx2, tab_rm)
    return out[:, :d].reshape(b, sq, d)


# speedup vs baseline: 1.6779x; 1.0013x over previous
import functools

import jax
import jax.numpy as jnp
from jax import lax
from jax.experimental import pallas as pl
from jax.experimental.pallas import tpu as pltpu
from jax.experimental.pallas import tpu_sc as plsc

NUM_WORKERS = 32
CHUNK = 128
NBUF = 8


def _gather_kernel(n_chunks, chunk, d, per_w):
    mesh = plsc.VectorSubcoreMesh(core_axis_name="c", subcore_axis_name="s")

    @functools.partial(
        pl.kernel,
        mesh=mesh,
        out_type=jax.ShapeDtypeStruct((NUM_WORKERS * per_w, 2 * d), jnp.float32),
        scratch_types=(
            [pltpu.VMEM((n_chunks, chunk), jnp.int32),
             pltpu.VMEM((NBUF, chunk, d), jnp.float32)]
            + [pltpu.SemaphoreType.DMA] * (2 * NBUF)
        ),
        compiler_params=pltpu.CompilerParams(use_tc_tiling_on_sc=False),
    )
    def emb(idx_hbm, tab_hbm, out_hbm, idx_v, rows_v, *sems):
        gsem = sems[:NBUF]
        wsem = sems[NBUF:]
        c = lax.axis_index("c")
        s = lax.axis_index("s")
        wid = s * 2 + c
        base = wid * per_w
        pltpu.sync_copy(idx_hbm.at[wid], idx_v)

        def fire_gather(slot, b):
            pltpu.async_copy(tab_hbm.at[idx_v.at[slot]], rows_v.at[b], gsem[b])

        def wait_gather(slot, b):
            pltpu.make_async_copy(
                tab_hbm.at[idx_v.at[slot]], rows_v.at[b], gsem[b]).wait()

        def fire_write(slot, b):
            pltpu.async_copy(
                rows_v.at[b],
                out_hbm.at[pl.ds(base + slot * chunk, chunk), pl.ds(0, d)],
                wsem[b])

        def wait_write(slot, b):
            pltpu.make_async_copy(
                rows_v.at[b],
                out_hbm.at[pl.ds(base + slot * chunk, chunk), pl.ds(0, d)],
                wsem[b]).wait()

        def do_slot(slot, k, fire, wait_prev):
            b = k % NBUF
            pb = (k - 1) % NBUF
            wait_gather(slot, b)
            fire_write(slot, b)
            if wait_prev:
                wait_write(slot - 1, pb)
            if fire:
                fire_gather(slot + NBUF - 1, pb)

        for j in range(NBUF - 1):
            fire_gather(j, j)

        for k in range(NBUF):
            do_slot(k, k, fire=(k + NBUF - 1 < n_chunks), wait_prev=(k >= 1))

        n_rounds = n_chunks // NBUF

        def body(r, _):
            s0 = r * NBUF
            for k in range(NBUF):
                do_slot(s0 + k, k, fire=True, wait_prev=True)
            return 0

        lax.fori_loop(1, n_rounds - 1, body, 0)

        s0 = (n_rounds - 1) * NBUF
        for k in range(NBUF):
            do_slot(s0 + k, k, fire=(s0 + k + NBUF - 1 < n_chunks),
                    wait_prev=True)

        wait_write(n_chunks - 1, (n_chunks - 1) % NBUF)

    return emb


TBLK = 32768
THALF = TBLK // 2


def _transpose_body(tt_ref, out_ref):
    d = tt_ref.shape[0]
    sub = 2048
    for j in range(THALF // sub):
        lo = j * sub
        y1 = jnp.swapaxes(tt_ref[:, pl.ds(lo, sub)], 0, 1)
        y2 = jnp.swapaxes(tt_ref[:, pl.ds(THALF + lo, sub)], 0, 1)
        out_ref[pl.ds(lo, sub), :] = jnp.concatenate([y1, y2], axis=1)


def _table_rowmajor(tt, v, d):
    grid = (v + TBLK - 1) // TBLK
    return pl.pallas_call(
        _transpose_body,
        grid=(grid,),
        in_specs=[pl.BlockSpec((d, TBLK), lambda j: (0, j))],
        out_specs=pl.BlockSpec((THALF, 2 * d), lambda j: (j, 0)),
        out_shape=jax.ShapeDtypeStruct((grid * THALF, 2 * d), jnp.float32),
        compiler_params=pltpu.CompilerParams(vmem_limit_bytes=100 * 1024 * 1024),
    )(tt)


def kernel(indices, table):
    b, sq = indices.shape
    v, d = table.shape
    n = b * sq
    per_w = n // NUM_WORKERS
    n_chunks = per_w // CHUNK
    idx = indices.astype(jnp.int32)
    t = idx % TBLK
    idx2 = (idx - t) + 2 * (t % THALF) + t // THALF
    idx2 = idx2.reshape(NUM_WORKERS, n_chunks, CHUNK)
    tab2 = _table_rowmajor(table.T, v, d)
    tab_rm = tab2.reshape(2 * tab2.shape[0], d)
    out = _gather_kernel(n_chunks, CHUNK, d, per_w)(idx2, tab_rm)
    return out[:, :d].reshape(b, sq, d)
